# baseline (speedup 1.0000x reference)
import functools

import jax
import jax.numpy as jnp
from jax import lax
from jax.experimental import pallas as pl
from jax.experimental.pallas import tpu as pltpu
from jax.experimental.pallas import tpu_sc as plsc

B, C, S = 1, 1024, 2048
E, H, K = 8, 2048, 2
T = S
TILE = 256
HC = 512
NT = (T * K + E * TILE) // TILE
P = NT * TILE

NC, NS, L = 2, 16, 16
NW = NC * NS


def _route_body(xt_ref, wg_ref, bg_ref, dest_ref, s0_ref, s1_ref,
                texp_ref, tcum_ref):
    x = xt_ref[...]
    wg = wg_ref[...]
    logits = lax.dot_general(
        x, wg, (((1,), (1,)), ((), ())),
        preferred_element_type=jnp.float32) + bg_ref[...][None, :]
    iota_e = lax.broadcasted_iota(jnp.int32, (T, E), 1)
    i1 = jnp.argmax(logits, axis=1)
    v1 = jnp.max(logits, axis=1)
    m1 = iota_e == i1[:, None]
    masked = jnp.where(m1, -jnp.inf, logits)
    i2 = jnp.argmax(masked, axis=1)
    v2 = jnp.max(masked, axis=1)
    m2 = iota_e == i2[:, None]
    e2 = jnp.exp(v2 - v1)
    s0 = 1.0 / (1.0 + e2)
    s1 = e2 / (1.0 + e2)
    s0_ref[...] = jnp.broadcast_to(s0[:, None], (T, L))
    s1_ref[...] = jnp.broadcast_to(s1[:, None], (T, L))

    oh = jnp.concatenate([m1, m2], axis=0).astype(jnp.float32)
    G, GR = 32, (2 * T) // 32
    ohg = oh.reshape(G, GR, E)
    iota_r = lax.broadcasted_iota(jnp.int32, (GR, GR), 0)
    iota_c = lax.broadcasted_iota(jnp.int32, (GR, GR), 1)
    ltri = (iota_c < iota_r).astype(jnp.float32)
    ltri_b = jnp.broadcast_to(ltri[None], (G, GR, GR))
    rank_w = lax.dot_general(
        ltri_b, ohg, (((2,), (1,)), ((0,), (0,))),
        preferred_element_type=jnp.float32)
    gsum = jnp.sum(ohg, axis=1)
    gi_r = lax.broadcasted_iota(jnp.int32, (G, G), 0)
    gi_c = lax.broadcasted_iota(jnp.int32, (G, G), 1)
    gtri = (gi_c < gi_r).astype(jnp.float32)
    gcum = lax.dot_general(
        gtri, gsum, (((1,), (0,)), ((), ())),
        preferred_element_type=jnp.float32)
    rank = (rank_w + gcum[:, None, :]).reshape(2 * T, E)

    counts = jnp.sum(gsum, axis=0).astype(jnp.int32)
    cap = ((counts + (TILE - 1)) // TILE) * TILE
    ei_r = lax.broadcasted_iota(jnp.int32, (E, E), 0)
    ei_c = lax.broadcasted_iota(jnp.int32, (E, E), 1)
    offs = jnp.sum(jnp.where(ei_c < ei_r, cap[None, :], 0), axis=1)
    offs_next = jnp.sum(jnp.where(ei_c <= ei_r, cap[None, :], 0), axis=1)

    dest = jnp.sum(oh * (offs[None, :].astype(jnp.float32) + rank), axis=1)
    dest_ref[...] = dest.astype(jnp.int32)

    jt = lax.iota(jnp.int32, NT) * TILE
    texp = jnp.sum((jt[:, None] >= offs_next[None, :]).astype(jnp.int32),
                   axis=1)
    texp_ref[...] = jnp.minimum(texp, E - 1)
    tcum_ref[...] = offs_next // TILE


def _route(xt, wg, bg):
    return pl.pallas_call(
        _route_body,
        out_shape=(
            jax.ShapeDtypeStruct((2 * T,), jnp.int32),
            jax.ShapeDtypeStruct((T, L), jnp.float32),
            jax.ShapeDtypeStruct((T, L), jnp.float32),
            jax.ShapeDtypeStruct((NT,), jnp.int32),
            jax.ShapeDtypeStruct((E,), jnp.int32),
        ),
    )(xt, wg, bg)


_RW = (2 * T) // NW
_CH = 32


@functools.cache
def _make_scatter_tokens():
    mesh = plsc.VectorSubcoreMesh(core_axis_name="c", subcore_axis_name="s")

    @functools.partial(
        pl.kernel,
        mesh=mesh,
        out_type=jax.ShapeDtypeStruct((P, C), jnp.float32),
        scratch_types=[
            pltpu.VMEM((_CH,), jnp.int32),
            pltpu.VMEM((_CH, C), jnp.float32),
            pltpu.SemaphoreType.DMA,
        ],
    )
    def _scatter_tokens(xt_hbm, dest_hbm, xs_hbm, idx_v, rows_v, sem):
        wid = lax.axis_index("s") * NC + lax.axis_index("c")
        base = wid * _RW
        for cidx in range(_RW // _CH):
            off = base + cidx * _CH
            tok = lax.rem(off, T)
            pltpu.sync_copy(dest_hbm.at[pl.ds(off, _CH)], idx_v)
            pltpu.sync_copy(xt_hbm.at[pl.ds(tok, _CH)], rows_v)
            pltpu.async_copy(rows_v, xs_hbm.at[idx_v], sem).wait()

    return _scatter_tokens


def _ffn_body(texp_ref, tcum_ref, xs_ref, w1_ref, b1_ref, w2_ref, b2_ref,
              ys_ref):
    j = pl.program_id(0)
    hc = pl.program_id(1)

    @pl.when(jnp.logical_and(j < tcum_ref[E - 1], hc == 0))
    def _init():
        ys_ref[...] = jnp.broadcast_to(b2_ref[0], (TILE, C))

    @pl.when(j < tcum_ref[E - 1])
    def _():
        x = xs_ref[...].astype(jnp.bfloat16)
        h = lax.dot_general(
            x, w1_ref[0].astype(jnp.bfloat16), (((1,), (1,)), ((), ())),
            preferred_element_type=jnp.float32) + b1_ref[0]
        h = jnp.maximum(h, 0.0).astype(jnp.bfloat16)
        ys_ref[...] += lax.dot_general(
            h, w2_ref[0].astype(jnp.bfloat16), (((1,), (1,)), ((), ())),
            preferred_element_type=jnp.float32)


def _grouped_ffn(xs, w1, b1, w2, b2, texp, tcum):
    grid_spec = pltpu.PrefetchScalarGridSpec(
        num_scalar_prefetch=2,
        grid=(NT, H // HC),
        in_specs=[
            pl.BlockSpec((TILE, C), lambda j, hc, texp, tcum: (j, 0)),
            pl.BlockSpec((1, HC, C), lambda j, hc, texp, tcum: (texp[j], hc, 0)),
            pl.BlockSpec((1, 1, HC), lambda j, hc, texp, tcum: (texp[j], 0, hc)),
            pl.BlockSpec((1, C, HC), lambda j, hc, texp, tcum: (texp[j], 0, hc)),
            pl.BlockSpec((1, 1, C), lambda j, hc, texp, tcum: (texp[j], 0, 0)),
        ],
        out_specs=pl.BlockSpec((TILE, C), lambda j, hc, texp, tcum: (j, 0)),
    )
    return pl.pallas_call(
        _ffn_body,
        grid_spec=grid_spec,
        out_shape=jax.ShapeDtypeStruct((P, C), jnp.float32),
    )(texp, tcum, xs, w1, b1.reshape(E, 1, H), w2, b2.reshape(E, 1, C))


_TW = T // NW
_CH2 = 16


@functools.cache
def _make_combine():
    mesh = plsc.VectorSubcoreMesh(core_axis_name="c", subcore_axis_name="s")

    @functools.partial(
        pl.kernel,
        mesh=mesh,
        out_type=jax.ShapeDtypeStruct((T, C), jnp.float32),
        scratch_types=[
            pltpu.VMEM((_CH2,), jnp.int32),
            pltpu.VMEM((_CH2,), jnp.int32),
            pltpu.VMEM((_CH2, C), jnp.float32),
            pltpu.VMEM((_CH2, C), jnp.float32),
            pltpu.VMEM((_CH2, L), jnp.float32),
            pltpu.VMEM((_CH2, L), jnp.float32),
            pltpu.VMEM((_CH2, C), jnp.float32),
            pltpu.SemaphoreType.DMA,
        ],
    )
    def _combine(ys_hbm, dest_hbm, s0_hbm, s1_hbm, out_hbm,
                 i0_v, i1_v, r0_v, r1_v, s0_v, s1_v, o_v, sem):
        wid = lax.axis_index("s") * NC + lax.axis_index("c")
        base = wid * _TW
        for cidx in range(_TW // _CH2):
            off = base + cidx * _CH2
            pltpu.sync_copy(dest_hbm.at[pl.ds(off, _CH2)], i0_v)
            pltpu.sync_copy(dest_hbm.at[pl.ds(off + T, _CH2)], i1_v)
            pltpu.sync_copy(s0_hbm.at[pl.ds(off, _CH2)], s0_v)
            pltpu.sync_copy(s1_hbm.at[pl.ds(off, _CH2)], s1_v)
            pltpu.async_copy(ys_hbm.at[i0_v], r0_v, sem).wait()
            pltpu.async_copy(ys_hbm.at[i1_v], r1_v, sem).wait()

            def body(t, _):
                w0 = s0_v[t]
                w1s = s1_v[t]
                for v in range(C // L):
                    sl = pl.ds(v * L, L)
                    o_v[t, sl] = w0 * r0_v[t, sl] + w1s * r1_v[t, sl]
                return 0

            lax.fori_loop(0, _CH2, body, 0)
            pltpu.sync_copy(o_v, out_hbm.at[pl.ds(off, _CH2)])

    return _combine


def kernel(x, Wg, bg, W1, b1, W2, b2):
    xt = jnp.transpose(x[0], (1, 0))
    dest, s0, s1, texp, tcum = _route(xt, Wg, bg)
    _STAGE = 4
    if _STAGE == 1:
        out = jnp.broadcast_to(s0[:, :1] + dest[:T, None].astype(jnp.float32), (T, C))
        return jnp.transpose(out, (1, 0))[None]
    xs = _make_scatter_tokens()(xt, dest)
    if _STAGE == 2:
        return jnp.transpose(xs[:T], (1, 0))[None]
    ys = _grouped_ffn(xs, W1, b1, W2, b2, texp, tcum)
    if _STAGE == 3:
        return jnp.transpose(ys[:T], (1, 0))[None]
    out = _make_combine()(ys, dest, s0, s1)
    return jnp.transpose(out, (1, 0))[None]


# speedup vs baseline: 1.0862x; 1.0862x over previous
import functools

import jax
import jax.numpy as jnp
from jax import lax
from jax.experimental import pallas as pl
from jax.experimental.pallas import tpu as pltpu
from jax.experimental.pallas import tpu_sc as plsc

B, C, S = 1, 1024, 2048
E, H, K = 8, 2048, 2
T = S
TILE = 128
NT = (T * K + E * TILE) // TILE
P = NT * TILE

NC, NS, L = 2, 16, 16
NW = NC * NS


def _route_body(xt_ref, wg_ref, bg_ref, dest_ref, s0_ref, s1_ref,
                texp_ref, tcum_ref):
    x = xt_ref[...]
    wg = wg_ref[...]
    logits = lax.dot_general(
        x, wg, (((1,), (1,)), ((), ())),
        preferred_element_type=jnp.float32) + bg_ref[...][None, :]
    iota_e = lax.broadcasted_iota(jnp.int32, (T, E), 1)
    i1 = jnp.argmax(logits, axis=1)
    v1 = jnp.max(logits, axis=1)
    m1 = iota_e == i1[:, None]
    masked = jnp.where(m1, -jnp.inf, logits)
    i2 = jnp.argmax(masked, axis=1)
    v2 = jnp.max(masked, axis=1)
    m2 = iota_e == i2[:, None]
    e2 = jnp.exp(v2 - v1)
    s0 = 1.0 / (1.0 + e2)
    s1 = e2 / (1.0 + e2)
    s0_ref[...] = jnp.broadcast_to(s0[:, None], (T, L))
    s1_ref[...] = jnp.broadcast_to(s1[:, None], (T, L))

    oh = jnp.concatenate([m1, m2], axis=0).astype(jnp.float32)
    G, GR = 32, (2 * T) // 32
    ohg = oh.reshape(G, GR, E)
    iota_r = lax.broadcasted_iota(jnp.int32, (GR, GR), 0)
    iota_c = lax.broadcasted_iota(jnp.int32, (GR, GR), 1)
    ltri = (iota_c < iota_r).astype(jnp.float32)
    ltri_b = jnp.broadcast_to(ltri[None], (G, GR, GR))
    rank_w = lax.dot_general(
        ltri_b, ohg, (((2,), (1,)), ((0,), (0,))),
        preferred_element_type=jnp.float32)
    gsum = jnp.sum(ohg, axis=1)
    gi_r = lax.broadcasted_iota(jnp.int32, (G, G), 0)
    gi_c = lax.broadcasted_iota(jnp.int32, (G, G), 1)
    gtri = (gi_c < gi_r).astype(jnp.float32)
    gcum = lax.dot_general(
        gtri, gsum, (((1,), (0,)), ((), ())),
        preferred_element_type=jnp.float32)
    rank = (rank_w + gcum[:, None, :]).reshape(2 * T, E)

    counts = jnp.sum(gsum, axis=0).astype(jnp.int32)
    cap = ((counts + (TILE - 1)) // TILE) * TILE
    ei_r = lax.broadcasted_iota(jnp.int32, (E, E), 0)
    ei_c = lax.broadcasted_iota(jnp.int32, (E, E), 1)
    offs = jnp.sum(jnp.where(ei_c < ei_r, cap[None, :], 0), axis=1)
    offs_next = jnp.sum(jnp.where(ei_c <= ei_r, cap[None, :], 0), axis=1)

    dest = jnp.sum(oh * (offs[None, :].astype(jnp.float32) + rank), axis=1)
    dest_ref[...] = dest.astype(jnp.int32)

    jt = lax.iota(jnp.int32, NT) * TILE
    texp = jnp.sum((jt[:, None] >= offs_next[None, :]).astype(jnp.int32),
                   axis=1)
    texp_ref[...] = jnp.minimum(texp, E - 1)
    tcum_ref[...] = offs_next // TILE


def _route(xt, wg, bg):
    return pl.pallas_call(
        _route_body,
        out_shape=(
            jax.ShapeDtypeStruct((2 * T,), jnp.int32),
            jax.ShapeDtypeStruct((T, L), jnp.float32),
            jax.ShapeDtypeStruct((T, L), jnp.float32),
            jax.ShapeDtypeStruct((NT,), jnp.int32),
            jax.ShapeDtypeStruct((E,), jnp.int32),
        ),
    )(xt, wg, bg)


_RW = (2 * T) // NW
_CH = 32


@functools.cache
def _make_scatter_tokens():
    mesh = plsc.VectorSubcoreMesh(core_axis_name="c", subcore_axis_name="s")

    @functools.partial(
        pl.kernel,
        mesh=mesh,
        out_type=jax.ShapeDtypeStruct((P, C), jnp.float32),
        scratch_types=[
            pltpu.VMEM((_CH,), jnp.int32),
            pltpu.VMEM((_CH, C), jnp.float32),
            pltpu.SemaphoreType.DMA,
        ],
    )
    def _scatter_tokens(xt_hbm, dest_hbm, xs_hbm, idx_v, rows_v, sem):
        wid = lax.axis_index("s") * NC + lax.axis_index("c")
        base = wid * _RW
        for cidx in range(_RW // _CH):
            off = base + cidx * _CH
            tok = lax.rem(off, T)
            pltpu.sync_copy(dest_hbm.at[pl.ds(off, _CH)], idx_v)
            pltpu.sync_copy(xt_hbm.at[pl.ds(tok, _CH)], rows_v)
            pltpu.async_copy(rows_v, xs_hbm.at[idx_v], sem).wait()

    return _scatter_tokens


def _ffn_body(texp_ref, tcum_ref, xs_ref, w1_ref, b1_ref, w2_ref, b2_ref,
              ys_ref):
    j = pl.program_id(0)

    @pl.when(j < tcum_ref[E - 1])
    def _():
        x = xs_ref[...].astype(jnp.bfloat16)
        h = lax.dot_general(
            x, w1_ref[0].astype(jnp.bfloat16), (((1,), (1,)), ((), ())),
            preferred_element_type=jnp.float32) + b1_ref[0]
        h = jnp.maximum(h, 0.0).astype(jnp.bfloat16)
        y = lax.dot_general(
            h, w2_ref[0].astype(jnp.bfloat16), (((1,), (1,)), ((), ())),
            preferred_element_type=jnp.float32) + b2_ref[0]
        ys_ref[...] = y


def _grouped_ffn(xs, w1, b1, w2, b2, texp, tcum):
    grid_spec = pltpu.PrefetchScalarGridSpec(
        num_scalar_prefetch=2,
        grid=(NT,),
        in_specs=[
            pl.BlockSpec((TILE, C), lambda j, texp, tcum: (j, 0)),
            pl.BlockSpec((1, H, C), lambda j, texp, tcum: (texp[j], 0, 0)),
            pl.BlockSpec((1, 1, H), lambda j, texp, tcum: (texp[j], 0, 0)),
            pl.BlockSpec((1, C, H), lambda j, texp, tcum: (texp[j], 0, 0)),
            pl.BlockSpec((1, 1, C), lambda j, texp, tcum: (texp[j], 0, 0)),
        ],
        out_specs=pl.BlockSpec((TILE, C), lambda j, texp, tcum: (j, 0)),
    )
    return pl.pallas_call(
        _ffn_body,
        grid_spec=grid_spec,
        out_shape=jax.ShapeDtypeStruct((P, C), jnp.float32),
    )(texp, tcum, xs, w1, b1.reshape(E, 1, H), w2, b2.reshape(E, 1, C))


_TW = T // NW
_CH2 = 16


@functools.cache
def _make_combine():
    mesh = plsc.VectorSubcoreMesh(core_axis_name="c", subcore_axis_name="s")

    @functools.partial(
        pl.kernel,
        mesh=mesh,
        out_type=jax.ShapeDtypeStruct((T, C), jnp.float32),
        scratch_types=[
            pltpu.VMEM((_CH2,), jnp.int32),
            pltpu.VMEM((_CH2,), jnp.int32),
            pltpu.VMEM((_CH2, C), jnp.float32),
            pltpu.VMEM((_CH2, C), jnp.float32),
            pltpu.VMEM((_CH2, L), jnp.float32),
            pltpu.VMEM((_CH2, L), jnp.float32),
            pltpu.VMEM((_CH2, C), jnp.float32),
            pltpu.SemaphoreType.DMA,
        ],
    )
    def _combine(ys_hbm, dest_hbm, s0_hbm, s1_hbm, out_hbm,
                 i0_v, i1_v, r0_v, r1_v, s0_v, s1_v, o_v, sem):
        wid = lax.axis_index("s") * NC + lax.axis_index("c")
        base = wid * _TW
        for cidx in range(_TW // _CH2):
            off = base + cidx * _CH2
            pltpu.sync_copy(dest_hbm.at[pl.ds(off, _CH2)], i0_v)
            pltpu.sync_copy(dest_hbm.at[pl.ds(off + T, _CH2)], i1_v)
            pltpu.sync_copy(s0_hbm.at[pl.ds(off, _CH2)], s0_v)
            pltpu.sync_copy(s1_hbm.at[pl.ds(off, _CH2)], s1_v)
            pltpu.async_copy(ys_hbm.at[i0_v], r0_v, sem).wait()
            pltpu.async_copy(ys_hbm.at[i1_v], r1_v, sem).wait()

            def body(t, _):
                w0 = s0_v[t]
                w1s = s1_v[t]
                for v in range(C // L):
                    sl = pl.ds(v * L, L)
                    o_v[t, sl] = w0 * r0_v[t, sl] + w1s * r1_v[t, sl]
                return 0

            lax.fori_loop(0, _CH2, body, 0)
            pltpu.sync_copy(o_v, out_hbm.at[pl.ds(off, _CH2)])

    return _combine


def kernel(x, Wg, bg, W1, b1, W2, b2):
    xt = jnp.transpose(x[0], (1, 0))
    dest, s0, s1, texp, tcum = _route(xt, Wg, bg)
    _STAGE = 4
    if _STAGE == 1:
        out = jnp.broadcast_to(s0[:, :1] + dest[:T, None].astype(jnp.float32), (T, C))
        return jnp.transpose(out, (1, 0))[None]
    xs = _make_scatter_tokens()(xt, dest)
    if _STAGE == 2:
        return jnp.transpose(xs[:T], (1, 0))[None]
    ys = _grouped_ffn(xs, W1, b1, W2, b2, texp, tcum)
    if _STAGE == 3:
        return jnp.transpose(ys[:T], (1, 0))[None]
    out = _make_combine()(ys, dest, s0, s1)
    return jnp.transpose(out, (1, 0))[None]


# speedup vs baseline: 1.4838x; 1.3661x over previous
import functools

import jax
import jax.numpy as jnp
from jax import lax
from jax.experimental import pallas as pl
from jax.experimental.pallas import tpu as pltpu
from jax.experimental.pallas import tpu_sc as plsc

B, C, S = 1, 1024, 2048
E, H, K = 8, 2048, 2
T = S
TILE = 256
NT = (T * K + E * TILE) // TILE
P = NT * TILE

NC, NS, L = 2, 16, 16
NW = NC * NS


def _route_body(x_ref, wg_ref, bg_ref, dest_ref, s0_ref, s1_ref,
                texp_ref, tcum_ref):
    x = x_ref[...]
    wg = wg_ref[...]
    logits = lax.dot_general(
        wg, x, (((1,), (0,)), ((), ())),
        preferred_element_type=jnp.float32) + bg_ref[...][:, None]
    iota_e = lax.broadcasted_iota(jnp.int32, (E, T), 0)
    i1 = jnp.argmax(logits, axis=0)
    v1 = jnp.max(logits, axis=0)
    m1 = iota_e == i1[None, :]
    masked = jnp.where(m1, -jnp.inf, logits)
    i2 = jnp.argmax(masked, axis=0)
    v2 = jnp.max(masked, axis=0)
    m2 = iota_e == i2[None, :]
    e2 = jnp.exp(v2 - v1)
    s0 = 1.0 / (1.0 + e2)
    s1 = e2 / (1.0 + e2)
    s0_ref[...] = jnp.broadcast_to(s0[:, None], (T, L))
    s1_ref[...] = jnp.broadcast_to(s1[:, None], (T, L))

    oh = jnp.concatenate([m1, m2], axis=1).astype(jnp.float32)
    incl = oh
    sh = 1
    while sh < 2 * T:
        shifted = jnp.concatenate(
            [jnp.zeros((E, sh), jnp.float32), incl[:, : 2 * T - sh]], axis=1)
        incl = incl + shifted
        sh *= 2
    rank = incl - oh

    counts = incl[:, 2 * T - 1:].astype(jnp.int32)
    cap = ((counts + (TILE - 1)) // TILE) * TILE
    ei_r = lax.broadcasted_iota(jnp.int32, (E, E), 0)
    ei_c = lax.broadcasted_iota(jnp.int32, (E, E), 1)
    cap_row = jnp.broadcast_to(jnp.transpose(cap, (1, 0)), (E, E))
    offs = jnp.sum(jnp.where(ei_c < ei_r, cap_row, 0), axis=1,
                   keepdims=True)
    offs_next = jnp.sum(jnp.where(ei_c <= ei_r, cap_row, 0), axis=1,
                        keepdims=True)

    dest = jnp.sum(oh * (offs.astype(jnp.float32) + rank), axis=0)
    dest_ref[...] = dest.astype(jnp.int32)

    jt = lax.iota(jnp.int32, NT) * TILE
    texp = jnp.sum((jt[None, :] >= offs_next).astype(jnp.int32), axis=0)
    texp_ref[...] = jnp.minimum(texp, E - 1)
    tcum_ref[...] = (offs_next // TILE).reshape(E)


def _route(x2d, wg, bg):
    return pl.pallas_call(
        _route_body,
        out_shape=(
            jax.ShapeDtypeStruct((2 * T,), jnp.int32),
            jax.ShapeDtypeStruct((T, L), jnp.float32),
            jax.ShapeDtypeStruct((T, L), jnp.float32),
            jax.ShapeDtypeStruct((NT,), jnp.int32),
            jax.ShapeDtypeStruct((E,), jnp.int32),
        ),
    )(x2d, wg, bg)


_RW = (2 * T) // NW
_CH = 32


@functools.cache
def _make_scatter_tokens():
    mesh = plsc.VectorSubcoreMesh(core_axis_name="c", subcore_axis_name="s")

    @functools.partial(
        pl.kernel,
        mesh=mesh,
        out_type=jax.ShapeDtypeStruct((P, C), jnp.float32),
        scratch_types=[
            pltpu.VMEM((_CH,), jnp.int32),
            pltpu.VMEM((_CH, C), jnp.float32),
            pltpu.SemaphoreType.DMA,
        ],
    )
    def _scatter_tokens(xt_hbm, dest_hbm, xs_hbm, idx_v, rows_v, sem):
        wid = lax.axis_index("s") * NC + lax.axis_index("c")
        base = wid * _RW
        for cidx in range(_RW // _CH):
            off = base + cidx * _CH
            tok = lax.rem(off, T)
            pltpu.sync_copy(dest_hbm.at[pl.ds(off, _CH)], idx_v)
            pltpu.sync_copy(xt_hbm.at[pl.ds(tok, _CH)], rows_v)
            pltpu.async_copy(rows_v, xs_hbm.at[idx_v], sem).wait()

    return _scatter_tokens


def _ffn_body(texp_ref, tcum_ref, xs_ref, w1_ref, b1_ref, w2_ref, b2_ref,
              ys_ref):
    j = pl.program_id(0)

    @pl.when(j < tcum_ref[E - 1])
    def _():
        x = xs_ref[...].astype(jnp.bfloat16)
        h = lax.dot_general(
            x, w1_ref[0].astype(jnp.bfloat16), (((1,), (1,)), ((), ())),
            preferred_element_type=jnp.float32) + b1_ref[0]
        h = jnp.maximum(h, 0.0).astype(jnp.bfloat16)
        y = lax.dot_general(
            h, w2_ref[0].astype(jnp.bfloat16), (((1,), (1,)), ((), ())),
            preferred_element_type=jnp.float32) + b2_ref[0]
        ys_ref[...] = y


def _grouped_ffn(xs, w1, b1, w2, b2, texp, tcum):
    grid_spec = pltpu.PrefetchScalarGridSpec(
        num_scalar_prefetch=2,
        grid=(NT,),
        in_specs=[
            pl.BlockSpec((TILE, C), lambda j, texp, tcum: (j, 0)),
            pl.BlockSpec((1, H, C), lambda j, texp, tcum: (texp[j], 0, 0)),
            pl.BlockSpec((1, 1, H), lambda j, texp, tcum: (texp[j], 0, 0)),
            pl.BlockSpec((1, C, H), lambda j, texp, tcum: (texp[j], 0, 0)),
            pl.BlockSpec((1, 1, C), lambda j, texp, tcum: (texp[j], 0, 0)),
        ],
        out_specs=pl.BlockSpec((TILE, C), lambda j, texp, tcum: (j, 0)),
    )
    return pl.pallas_call(
        _ffn_body,
        grid_spec=grid_spec,
        out_shape=jax.ShapeDtypeStruct((P, C), jnp.float32),
    )(texp, tcum, xs, w1, b1.reshape(E, 1, H), w2, b2.reshape(E, 1, C))


_TW = T // NW
_CH2 = 16


@functools.cache
def _make_combine():
    mesh = plsc.VectorSubcoreMesh(core_axis_name="c", subcore_axis_name="s")

    @functools.partial(
        pl.kernel,
        mesh=mesh,
        out_type=jax.ShapeDtypeStruct((T, C), jnp.float32),
        scratch_types=[
            pltpu.VMEM((_CH2,), jnp.int32),
            pltpu.VMEM((_CH2,), jnp.int32),
            pltpu.VMEM((_CH2, C), jnp.float32),
            pltpu.VMEM((_CH2, C), jnp.float32),
            pltpu.VMEM((_CH2, L), jnp.float32),
            pltpu.VMEM((_CH2, L), jnp.float32),
            pltpu.VMEM((_CH2, C), jnp.float32),
            pltpu.SemaphoreType.DMA,
        ],
    )
    def _combine(ys_hbm, dest_hbm, s0_hbm, s1_hbm, out_hbm,
                 i0_v, i1_v, r0_v, r1_v, s0_v, s1_v, o_v, sem):
        wid = lax.axis_index("s") * NC + lax.axis_index("c")
        base = wid * _TW
        for cidx in range(_TW // _CH2):
            off = base + cidx * _CH2
            pltpu.sync_copy(dest_hbm.at[pl.ds(off, _CH2)], i0_v)
            pltpu.sync_copy(dest_hbm.at[pl.ds(off + T, _CH2)], i1_v)
            pltpu.sync_copy(s0_hbm.at[pl.ds(off, _CH2)], s0_v)
            pltpu.sync_copy(s1_hbm.at[pl.ds(off, _CH2)], s1_v)
            pltpu.async_copy(ys_hbm.at[i0_v], r0_v, sem).wait()
            pltpu.async_copy(ys_hbm.at[i1_v], r1_v, sem).wait()

            def body(t, _):
                w0 = s0_v[t]
                w1s = s1_v[t]
                for v in range(C // L):
                    sl = pl.ds(v * L, L)
                    o_v[t, sl] = w0 * r0_v[t, sl] + w1s * r1_v[t, sl]
                return 0

            lax.fori_loop(0, _CH2, body, 0)
            pltpu.sync_copy(o_v, out_hbm.at[pl.ds(off, _CH2)])

    return _combine


def kernel(x, Wg, bg, W1, b1, W2, b2):
    xt = jnp.transpose(x[0], (1, 0))
    dest, s0, s1, texp, tcum = _route(x[0], Wg, bg)
    xs = _make_scatter_tokens()(xt, dest)
    ys = _grouped_ffn(xs, W1, b1, W2, b2, texp, tcum)
    out = _make_combine()(ys, dest, s0, s1)
    return jnp.transpose(out, (1, 0))[None]
